# baseline (speedup 1.0000x reference)
import functools

import jax
import jax.numpy as jnp
from jax import lax
from jax.experimental import pallas as pl
from jax.experimental.pallas import tpu as pltpu
from jax.experimental.pallas import tpu_sc as plsc

_NC = 2
_NS = 16
_NW = _NC * _NS
_NSLOT = 2


@functools.lru_cache(maxsize=None)
def _sc_edge_diff(E, N, D):
    epw = E // _NW
    C = 40
    n_chunks = epw // C
    stripe = (N // _NS) // 8 * 8
    last_stripe = N - (_NS - 1) * stripe

    mesh = plsc.VectorSubcoreMesh(core_axis_name="c", subcore_axis_name="s")

    @functools.partial(
        pl.kernel,
        mesh=mesh,
        out_type=jax.ShapeDtypeStruct((E, D), jnp.float32),
        scratch_types=[
            pltpu.VMEM((epw,), jnp.int32),
            pltpu.VMEM((epw,), jnp.int32),
            pltpu.VMEM((_NSLOT, C, D), jnp.float32),
            pltpu.VMEM((_NSLOT, C, D), jnp.float32),
            pltpu.VMEM((_NSLOT, C, D), jnp.float32),
            pltpu.VMEM_SHARED((N, D), jnp.float32),
            pltpu.SemaphoreType.DMA,
            pltpu.SemaphoreType.DMA,
        ] + [pltpu.SemaphoreType.DMA] * (3 * _NSLOT),
    )
    def k(x_hbm, src_hbm, dst_hbm, out_hbm,
          idx_s, idx_d, rows_s, rows_d, diff, x_sp, sem_is, sem_id, *sems):
        sem_gs = sems[0:_NSLOT]
        sem_gd = sems[_NSLOT:2 * _NSLOT]
        sem_o = sems[2 * _NSLOT:3 * _NSLOT]

        wid = lax.axis_index("s") * _NC + lax.axis_index("c")
        base = wid * epw

        cp_is = pltpu.async_copy(src_hbm.at[pl.ds(base, epw)], idx_s, sem_is)
        cp_id = pltpu.async_copy(dst_hbm.at[pl.ds(base, epw)], idx_d, sem_id)

        sid = lax.axis_index("s")

        @pl.when(sid < _NS - 1)
        def _():
            pltpu.sync_copy(x_hbm.at[pl.ds(sid * stripe, stripe)],
                            x_sp.at[pl.ds(sid * stripe, stripe)])

        @pl.when(sid == _NS - 1)
        def _():
            pltpu.sync_copy(x_hbm.at[pl.ds((_NS - 1) * stripe, last_stripe)],
                            x_sp.at[pl.ds((_NS - 1) * stripe, last_stripe)])

        plsc.subcore_barrier()

        cp_is.wait()
        cp_id.wait()

        def issue_gathers(c, b):
            pltpu.async_copy(
                x_sp.at[idx_s.at[pl.ds(c * C, C)]], rows_s.at[b], sem_gs[b])
            pltpu.async_copy(
                x_hbm.at[idx_d.at[pl.ds(c * C, C)]], rows_d.at[b], sem_gd[b])

        def wait_gathers(b):
            pltpu.make_async_copy(
                x_sp.at[idx_s.at[pl.ds(0, C)]], rows_s.at[b], sem_gs[b]).wait()
            pltpu.make_async_copy(
                x_hbm.at[idx_d.at[pl.ds(0, C)]], rows_d.at[b], sem_gd[b]).wait()

        def issue_write(c, b):
            pltpu.async_copy(
                diff.at[b], out_hbm.at[pl.ds(base + c * C, C)], sem_o[b])

        def wait_write(b):
            pltpu.make_async_copy(
                diff.at[b], out_hbm.at[pl.ds(base, C)], sem_o[b]).wait()

        def compute(b):
            RU = 4

            def row_body(r, rcarry):
                for rr in range(RU):
                    row = r * RU + rr
                    for v in range(D // 16):
                        sl = pl.ds(v * 16, 16)
                        diff[b, row, sl] = rows_d[b, row, sl] - rows_s[b, row, sl]
                return rcarry
            lax.fori_loop(0, C // RU, row_body, 0)

        def stage(c, b):
            wait_gathers(b)

            @pl.when(c >= _NSLOT)
            def _():
                wait_write(b)

            compute(b)
            issue_write(c, b)

            @pl.when(c + _NSLOT < n_chunks)
            def _():
                issue_gathers(c + _NSLOT, b)

        for b in range(min(_NSLOT, n_chunks)):
            issue_gathers(b, b)

        n_pipe = n_chunks - (n_chunks % _NSLOT)

        def body(i, carry):
            for b in range(_NSLOT):
                stage(i * _NSLOT + b, b)
            return carry

        lax.fori_loop(0, n_pipe // _NSLOT, body, 0)
        for c in range(n_pipe, n_chunks):
            stage(c, c % _NSLOT)

        for b in range(min(_NSLOT, n_chunks)):
            wait_write(b)

    return k


def kernel(x, edge_index):
    N, D = x.shape
    E = edge_index.shape[1]
    src = edge_index[0]
    dst = edge_index[1]
    return _sc_edge_diff(E, N, D)(x, src, dst)


# speedup vs baseline: 1.2828x; 1.2828x over previous
import functools

import jax
import jax.numpy as jnp
from jax import lax
from jax.experimental import pallas as pl
from jax.experimental.pallas import tpu as pltpu
from jax.experimental.pallas import tpu_sc as plsc

_NC = 2
_NS = 16
_NW = _NC * _NS
_NSLOT = 2


@functools.lru_cache(maxsize=None)
def _sc_edge_diff(E, N, D):
    epw = E // _NW
    C = 40
    n_chunks = epw // C
    stripe = (N // _NS) // 8 * 8
    last_stripe = N - (_NS - 1) * stripe

    mesh = plsc.VectorSubcoreMesh(core_axis_name="c", subcore_axis_name="s")

    @functools.partial(
        pl.kernel,
        mesh=mesh,
        out_type=jax.ShapeDtypeStruct((E, D), jnp.float32),
        scratch_types=[
            pltpu.VMEM((epw,), jnp.int32),
            pltpu.VMEM((epw,), jnp.int32),
            pltpu.VMEM((_NSLOT, C, D), jnp.float32),
            pltpu.VMEM((_NSLOT, C, D), jnp.float32),
            pltpu.VMEM((_NSLOT, C, D), jnp.float32),
            pltpu.VMEM_SHARED((N, D), jnp.float32),
            pltpu.SemaphoreType.DMA,
            pltpu.SemaphoreType.DMA,
        ] + [pltpu.SemaphoreType.DMA] * (3 * _NSLOT),
    )
    def k(x_hbm, src_hbm, dst_hbm, out_hbm,
          idx_s, idx_d, rows_s, rows_d, diff, x_sp, sem_is, sem_id, *sems):
        sem_gs = sems[0:_NSLOT]
        sem_gd = sems[_NSLOT:2 * _NSLOT]
        sem_o = sems[2 * _NSLOT:3 * _NSLOT]

        wid = lax.axis_index("s") * _NC + lax.axis_index("c")
        base = wid * epw

        cp_is = pltpu.async_copy(src_hbm.at[pl.ds(base, epw)], idx_s, sem_is)
        cp_id = pltpu.async_copy(dst_hbm.at[pl.ds(base, epw)], idx_d, sem_id)

        sid = lax.axis_index("s")

        @pl.when(sid < _NS - 1)
        def _():
            pltpu.sync_copy(x_hbm.at[pl.ds(sid * stripe, stripe)],
                            x_sp.at[pl.ds(sid * stripe, stripe)])

        @pl.when(sid == _NS - 1)
        def _():
            pltpu.sync_copy(x_hbm.at[pl.ds((_NS - 1) * stripe, last_stripe)],
                            x_sp.at[pl.ds((_NS - 1) * stripe, last_stripe)])

        plsc.subcore_barrier()

        cp_is.wait()
        cp_id.wait()

        def issue_gathers(c, b):
            pltpu.async_copy(
                x_sp.at[idx_s.at[pl.ds(c * C, C)]], rows_s.at[b], sem_gs[b])
            pltpu.async_copy(
                x_sp.at[idx_d.at[pl.ds(c * C, C)]], rows_d.at[b], sem_gd[b])

        def wait_gathers(b):
            pltpu.make_async_copy(
                x_sp.at[idx_s.at[pl.ds(0, C)]], rows_s.at[b], sem_gs[b]).wait()
            pltpu.make_async_copy(
                x_sp.at[idx_d.at[pl.ds(0, C)]], rows_d.at[b], sem_gd[b]).wait()

        def issue_write(c, b):
            pltpu.async_copy(
                diff.at[b], out_hbm.at[pl.ds(base + c * C, C)], sem_o[b])

        def wait_write(b):
            pltpu.make_async_copy(
                diff.at[b], out_hbm.at[pl.ds(base, C)], sem_o[b]).wait()

        def compute(b):
            RU = 4

            def row_body(r, rcarry):
                for rr in range(RU):
                    row = r * RU + rr
                    for v in range(D // 16):
                        sl = pl.ds(v * 16, 16)
                        diff[b, row, sl] = rows_d[b, row, sl] - rows_s[b, row, sl]
                return rcarry
            lax.fori_loop(0, C // RU, row_body, 0)

        def stage(c, b):
            wait_gathers(b)

            @pl.when(c >= _NSLOT)
            def _():
                wait_write(b)

            compute(b)
            issue_write(c, b)

            @pl.when(c + _NSLOT < n_chunks)
            def _():
                issue_gathers(c + _NSLOT, b)

        for b in range(min(_NSLOT, n_chunks)):
            issue_gathers(b, b)

        n_pipe = n_chunks - (n_chunks % _NSLOT)

        def body(i, carry):
            for b in range(_NSLOT):
                stage(i * _NSLOT + b, b)
            return carry

        lax.fori_loop(0, n_pipe // _NSLOT, body, 0)
        for c in range(n_pipe, n_chunks):
            stage(c, c % _NSLOT)

        for b in range(min(_NSLOT, n_chunks)):
            wait_write(b)

    return k


def kernel(x, edge_index):
    N, D = x.shape
    E = edge_index.shape[1]
    src = edge_index[0]
    dst = edge_index[1]
    return _sc_edge_diff(E, N, D)(x, src, dst)
